# Initial kernel scaffold; baseline (speedup 1.0000x reference)
import jax
import jax.numpy as jnp
from jax.experimental import pallas as pl


def kernel(x, edge_index, edge_attr, coords, bc_disp, bc_rot, Wn1, bn1, Wn2, bn2, We1, be1, We2, be2, Wmsg, bmsg, Wnode, bnode, Wd1, bd1, Wd2, bd2, Wd3, bd3, Wd4, bd4):
    raise NotImplementedError("write your pallas kernel here")


# speedup vs baseline: 2.4223x; 2.4223x over previous
import functools

import jax
import jax.numpy as jnp
from jax import lax
from jax.experimental import pallas as pl
from jax.experimental.pallas import tpu as pltpu
from jax.experimental.pallas import tpu_sc as plsc

_mm = functools.partial(jnp.dot, precision=jax.lax.Precision.HIGHEST)

_NC = 2
_NS = 16
_CH = 128


def _enc_body(x_ref, w1, b1, w2, b2, wa, wb, h_ref, a_ref, b_ref):
    t = jnp.maximum(_mm(x_ref[...], w1[...]) + b1[...], 0.0)
    h = _mm(t, w2[...]) + b2[...]
    h_ref[...] = h
    a_ref[...] = _mm(h, wa[...])
    b_ref[...] = _mm(h, wb[...])


def _ew_body(ea_ref, w1, b1, w2, b2, w3, bm, o_ref):
    e = _mm(jnp.maximum(_mm(ea_ref[...], w1[...]) + b1[...], 0.0), w2[...]) + b2[...]
    for l in range(o_ref.shape[0]):
        o_ref[l] = _mm(e, w3[l]) + bm[l]


def _upd_ab_body(h_ref, agg_ref, wh, wa, bn, wan, wbn, o_ref, a_ref, b_ref):
    hb = h_ref[...]
    agg = agg_ref[0] + agg_ref[1]
    hn = hb + jnp.maximum(_mm(hb, wh[...]) + _mm(agg, wa[...]) + bn[...], 0.0)
    o_ref[...] = hn
    a_ref[...] = _mm(hn, wan[...])
    b_ref[...] = _mm(hn, wbn[...])


def _dec_body(h_ref, agg_ref, wh, wa, bn, c_ref, bcd_ref, bcr_ref,
              wc, wdh, b1, w2, b2, w3, b3, w4, b4, o_ref):
    hb = h_ref[...]
    agg = agg_ref[0] + agg_ref[1]
    hn = hb + jnp.maximum(_mm(hb, wh[...]) + _mm(agg, wa[...]) + bn[...], 0.0)
    z = jnp.maximum(_mm(c_ref[...], wc[...]) + _mm(hn, wdh[...]) + b1[...], 0.0)
    z = jnp.maximum(_mm(z, w2[...]) + b2[...], 0.0)
    z = jnp.maximum(_mm(z, w3[...]) + b3[...], 0.0)
    p = _mm(z, w4[...]) + b4[...]
    col = lax.broadcasted_iota(jnp.int32, p.shape, 1)
    fac = jnp.where(col < 2, 1.0 - bcd_ref[...], 1.0 - bcr_ref[...])
    o_ref[...] = p * fac


def _make_agg(e_pad, n_nodes, n_layers, l, h_dim):
    tpt = e_pad // (_CH * _NC * _NS)
    nd = ((n_nodes + 1 + _NS * 8 - 1) // (_NS * 8)) * (_NS * 8)
    mesh = plsc.VectorSubcoreMesh(core_axis_name="c", subcore_axis_name="s",
                                  num_cores=_NC, num_subcores=_NS)

    @functools.partial(
        pl.kernel,
        out_type=jax.ShapeDtypeStruct((_NC, n_nodes, h_dim), jnp.float32),
        mesh=mesh,
        scratch_types=[
            pltpu.VMEM((_CH,), jnp.int32),
            pltpu.VMEM((_CH,), jnp.int32),
            pltpu.VMEM((_CH, h_dim), jnp.float32),
            pltpu.VMEM((_CH, h_dim), jnp.float32),
            pltpu.VMEM((_CH, h_dim), jnp.float32),
            pltpu.VMEM_SHARED((nd, h_dim), jnp.float32),
            pltpu.SemaphoreType.DMA,
            pltpu.SemaphoreType.DMA,
            pltpu.SemaphoreType.DMA,
        ],
    )
    def agg_kernel(a_hbm, b_hbm, ew_hbm, src_hbm, dst_hbm, out_hbm,
                   src_v, dst_v, abuf, bbuf, ebuf, agg_sh, sem_a, sem_b, sem_e):
        c = lax.axis_index("c")
        s = lax.axis_index("s")
        nvec = h_dim // 16
        zero = jnp.zeros((16,), jnp.float32)

        def zrow(r, carry):
            for j in range(nvec):
                ebuf[r, pl.ds(j * 16, 16)] = zero
            return carry
        lax.fori_loop(0, _CH, zrow, 0)

        rows_per_tile = nd // _NS
        zbase = s * rows_per_tile
        nfull = rows_per_tile // _CH
        rem = rows_per_tile - nfull * _CH
        for k in range(nfull):
            pltpu.sync_copy(ebuf, agg_sh.at[pl.ds(zbase + k * _CH, _CH)])
        if rem:
            pltpu.sync_copy(ebuf.at[pl.ds(0, rem)],
                            agg_sh.at[pl.ds(zbase + nfull * _CH, rem)])
        plsc.subcore_barrier()

        tile_chunk0 = (c * _NS + s) * tpt
        ew_base = l * e_pad

        def chunk(t, carry):
            e0 = (tile_chunk0 + t) * _CH
            pltpu.sync_copy(src_hbm.at[pl.ds(e0, _CH)], src_v)
            pltpu.sync_copy(dst_hbm.at[pl.ds(e0, _CH)], dst_v)
            ca = pltpu.async_copy(a_hbm.at[src_v], abuf, sem_a)
            cb = pltpu.async_copy(b_hbm.at[dst_v], bbuf, sem_b)
            ce = pltpu.async_copy(ew_hbm.at[pl.ds(ew_base + e0, _CH)], ebuf, sem_e)
            ca.wait()
            cb.wait()
            ce.wait()

            def row(r, cr):
                for j in range(nvec):
                    sl = pl.ds(j * 16, 16)
                    ebuf[r, sl] = jnp.maximum(
                        abuf[r, sl] + bbuf[r, sl] + ebuf[r, sl], 0.0)
                return cr
            lax.fori_loop(0, _CH, row, 0)

            pltpu.sync_copy(ebuf, agg_sh.at[dst_v], add=True)
            return carry
        lax.fori_loop(0, tpt, chunk, 0)
        plsc.subcore_barrier()

        out_rows = (n_nodes // _NS) // 8 * 8
        tail = n_nodes - out_rows * _NS
        r0 = s * out_rows
        pltpu.sync_copy(agg_sh.at[pl.ds(r0, out_rows)],
                        out_hbm.at[c].at[pl.ds(r0, out_rows)])
        if tail:
            @pl.when(s == _NS - 1)
            def _():
                t0 = out_rows * _NS
                pltpu.sync_copy(agg_sh.at[pl.ds(t0, tail)],
                                out_hbm.at[c].at[pl.ds(t0, tail)])

    return agg_kernel


def kernel(x, edge_index, edge_attr, coords, bc_disp, bc_rot,
           Wn1, bn1, Wn2, bn2, We1, be1, We2, be2,
           Wmsg, bmsg, Wnode, bnode,
           Wd1, bd1, Wd2, bd2, Wd3, bd3, Wd4, bd4):
    n = x.shape[0]
    e_cnt = edge_index.shape[1]
    nl = Wmsg.shape[0]
    h_dim = Wn1.shape[1]

    ept = _CH * _NC * _NS
    e_pad = ((e_cnt + ept - 1) // ept) * ept
    pad = e_pad - e_cnt
    src_p = jnp.concatenate([edge_index[0], jnp.zeros((pad,), jnp.int32)])
    dst_p = jnp.concatenate([edge_index[1], jnp.full((pad,), n, jnp.int32)])
    ea_p = jnp.concatenate(
        [edge_attr, jnp.zeros((pad, edge_attr.shape[1]), edge_attr.dtype)], axis=0)

    b2d = lambda v: v.reshape(1, -1)
    W1 = Wmsg[:, :h_dim, :]
    W2 = Wmsg[:, h_dim:2 * h_dim, :]
    W3 = Wmsg[:, 2 * h_dim:, :]
    Wh = Wnode[:, :h_dim, :]
    Wa = Wnode[:, h_dim:, :]

    f32 = jnp.float32
    nhs = jax.ShapeDtypeStruct((n, h_dim), f32)

    h, A, B = pl.pallas_call(_enc_body, out_shape=(nhs, nhs, nhs))(
        x, Wn1, b2d(bn1), Wn2, b2d(bn2), W1[0], W2[0])

    eblk = 2048
    ein = edge_attr.shape[1]
    ew = pl.pallas_call(
        _ew_body,
        grid=(e_pad // eblk,),
        in_specs=[
            pl.BlockSpec((eblk, ein), lambda i: (i, 0)),
            pl.BlockSpec((ein, h_dim), lambda i: (0, 0)),
            pl.BlockSpec((1, h_dim), lambda i: (0, 0)),
            pl.BlockSpec((h_dim, h_dim), lambda i: (0, 0)),
            pl.BlockSpec((1, h_dim), lambda i: (0, 0)),
            pl.BlockSpec((nl, h_dim, h_dim), lambda i: (0, 0, 0)),
            pl.BlockSpec((nl, h_dim), lambda i: (0, 0)),
        ],
        out_specs=pl.BlockSpec((nl, eblk, h_dim), lambda i: (0, i, 0)),
        out_shape=jax.ShapeDtypeStruct((nl, e_pad, h_dim), f32),
    )(ea_p, We1, b2d(be1), We2, b2d(be2), W3, bmsg)
    ew_flat = ew.reshape(nl * e_pad, h_dim)

    pred = None
    for l in range(nl):
        aggp = _make_agg(e_pad, n, nl, l, h_dim)(A, B, ew_flat, src_p, dst_p)
        if l + 1 < nl:
            h, A, B = pl.pallas_call(_upd_ab_body, out_shape=(nhs, nhs, nhs))(
                h, aggp, Wh[l], Wa[l], b2d(bnode[l]), W1[l + 1], W2[l + 1])
        else:
            pred = pl.pallas_call(
                _dec_body, out_shape=jax.ShapeDtypeStruct((n, 3), f32))(
                h, aggp, Wh[l], Wa[l], b2d(bnode[l]),
                coords, bc_disp, bc_rot,
                Wd1[:3], Wd1[3:], b2d(bd1), Wd2, b2d(bd2),
                Wd3, b2d(bd3), Wd4, b2d(bd4))
    return pred
